# Initial kernel scaffold; baseline (speedup 1.0000x reference)
import jax
import jax.numpy as jnp
from jax.experimental import pallas as pl


def kernel(bw, tr, p_lidx, mask, params):
    raise NotImplementedError("write your pallas kernel here")


# speedup vs baseline: 2.3846x; 2.3846x over previous
import functools

import jax
import jax.numpy as jnp
from jax import lax
from jax.experimental import pallas as pl
from jax.experimental.pallas import tpu as pltpu
from jax.experimental.pallas import tpu_sc as plsc

DIM = 128
L = 20
S = 21
NP = 16384
NL = 2048
NB = 4
TAB = 2056
ACC = 2064
E = NP * L
P = 256
NPB = NP // P
PH = 2048


def _ln(x, g, b, eps=1e-5):
    m = jnp.mean(x, axis=-1, keepdims=True)
    v = jnp.mean((x - m) ** 2, axis=-1, keepdims=True)
    return (x - m) * lax.rsqrt(v + eps) * g + b


def _gelu(x):
    return 0.5 * x * (1.0 + lax.erf(x * 0.7071067811865476))


def _full(shape):
    return pl.BlockSpec(shape, lambda i: tuple(0 for _ in shape))


def _embed_body(bw_ref, tr_ref, wbw_ref, bbw_ref, wtr_ref, btr_ref,
                lemb_ref, pst_ref):
    lemb_ref[...] = bw_ref[...] * wbw_ref[...] + bbw_ref[...]
    pst_ref[...] = tr_ref[...] * wtr_ref[...] + btr_ref[...]


def _embed(bw, tr, params):
    bw2 = bw.reshape(NL, 1)
    tr2 = tr.reshape(NP, 1)
    pe, te = params["bw_emb"], params["tr_emb"]
    return pl.pallas_call(
        _embed_body,
        grid=(8,),
        in_specs=[
            pl.BlockSpec((NL // 8, 1), lambda i: (i, 0)),
            pl.BlockSpec((NP // 8, 1), lambda i: (i, 0)),
            _full((1, DIM)), _full((1, DIM)), _full((1, DIM)), _full((1, DIM)),
        ],
        out_specs=[
            pl.BlockSpec((NL // 8, DIM), lambda i: (i, 0)),
            pl.BlockSpec((NP // 8, DIM), lambda i: (i, 0)),
        ],
        out_shape=[
            jax.ShapeDtypeStruct((NL, DIM), jnp.float32),
            jax.ShapeDtypeStruct((NP, DIM), jnp.float32),
        ],
    )(bw2, tr2, pe["w"], pe["b"][None, :], te["w"], te["b"][None, :])


def _path_body(pst_ref, g_ref, mask_ref,
               wc1_ref, bc1_ref, wc2_ref, bc2_ref,
               g1_ref, d1_ref, g2_ref, d2_ref, g3_ref, d3_ref,
               wl1_ref, bl1_ref, wl2_ref, bl2_ref,
               wpw_ref, bpw_ref,
               pst_out_ref, pbw_out_ref):
    x = jnp.concatenate([pst_ref[...][None], g_ref[...]], axis=0)
    mk = mask_ref[0][:, :, None]
    h = _ln(x * mk, g1_ref[...], d1_ref[...])
    h2 = h.reshape(S, P * DIM)
    hc = jnp.dot(wc1_ref[...], h2, preferred_element_type=jnp.float32)
    hc = _gelu(hc + bc1_ref[...])
    hc = jnp.dot(wc2_ref[...], hc, preferred_element_type=jnp.float32)
    hc = hc + bc2_ref[...]
    x = x + hc.reshape(S, P, DIM)
    h = _ln(x * mk, g2_ref[...], d2_ref[...]).reshape(S * P, DIM)
    h = _gelu(jnp.dot(h, wl1_ref[...], preferred_element_type=jnp.float32)
              + bl1_ref[...])
    h = jnp.dot(h, wl2_ref[...], preferred_element_type=jnp.float32) + bl2_ref[...]
    x = x + h.reshape(S, P, DIM)
    x = _ln(x, g3_ref[...], d3_ref[...])
    p = x[0]
    pst_out_ref[...] = p
    t = jnp.sum(p * wpw_ref[...], axis=-1, keepdims=True) + bpw_ref[...]
    pw = 1.0 / (1.0 + jnp.exp(-t))
    pbw_out_ref[...] = x[1:] * pw[None]


def _path_stage(pst, g3, mask3, pp, pw_p):
    ch = S // 2
    return pl.pallas_call(
        _path_body,
        grid=(NPB,),
        in_specs=[
            pl.BlockSpec((P, DIM), lambda i: (i, 0)),
            pl.BlockSpec((L, P, DIM), lambda i: (0, i, 0)),
            pl.BlockSpec((1, S, P), lambda i: (i, 0, 0)),
            _full((ch, S)), _full((ch, 1)), _full((S, ch)), _full((S, 1)),
            _full((DIM,)), _full((DIM,)), _full((DIM,)), _full((DIM,)),
            _full((DIM,)), _full((DIM,)),
            _full((DIM, DIM // 2)), _full((DIM // 2,)),
            _full((DIM // 2, DIM)), _full((DIM,)),
            _full((1, DIM)), _full((1, 1)),
        ],
        out_specs=[
            pl.BlockSpec((P, DIM), lambda i: (i, 0)),
            pl.BlockSpec((L, P, DIM), lambda i: (0, i, 0)),
        ],
        out_shape=[
            jax.ShapeDtypeStruct((NP, DIM), jnp.float32),
            jax.ShapeDtypeStruct((L, NP, DIM), jnp.float32),
        ],
    )(pst, g3, mask3,
      pp["cu_c1"]["w"], pp["cu_c1"]["b"][:, None],
      pp["cu_c2"]["w"], pp["cu_c2"]["b"][:, None],
      pp["cu_ln"]["g"], pp["cu_ln"]["b"],
      pp["iu_ln"]["g"], pp["iu_ln"]["b"],
      pp["norm"]["g"], pp["norm"]["b"],
      pp["iu_l1"]["w"], pp["iu_l1"]["b"],
      pp["iu_l2"]["w"], pp["iu_l2"]["b"],
      pw_p["w"].T, pw_p["b"][None, :])


def _link_body(part_ref, lold_ref,
               ng_ref, nd_ref,
               wc1_ref, bc1_ref, wc2_ref, bc2_ref,
               g1_ref, d1_ref, g2_ref, d2_ref, g3_ref, d3_ref,
               wl1_ref, bl1_ref, wl2_ref, bl2_ref,
               out_ref):
    tab = part_ref[0] + part_ref[1]
    link = _ln(tab, ng_ref[...], nd_ref[...])
    x = jnp.concatenate([lold_ref[...], link], axis=-1)
    h = _ln(x, g1_ref[...], d1_ref[...])
    hc = _gelu(jnp.dot(wc1_ref[...], h, preferred_element_type=jnp.float32)
               + bc1_ref[...])
    hc = jnp.dot(wc2_ref[...], hc, preferred_element_type=jnp.float32) + bc2_ref[...]
    x = x + hc
    h = _ln(x, g2_ref[...], d2_ref[...])
    h = _gelu(jnp.dot(h, wl1_ref[...], preferred_element_type=jnp.float32)
              + bl1_ref[...])
    h = jnp.dot(h, wl2_ref[...], preferred_element_type=jnp.float32) + bl2_ref[...]
    x = x + h
    x = _ln(x, g3_ref[...], d3_ref[...])
    j = lax.broadcasted_iota(jnp.int32, (2 * DIM, DIM), 0)
    k = lax.broadcasted_iota(jnp.int32, (2 * DIM, DIM), 1)
    sel_ev = (j == 2 * k).astype(jnp.float32)
    sel_od = (j == 2 * k + 1).astype(jnp.float32)
    ev = jnp.dot(x, sel_ev, preferred_element_type=jnp.float32)
    od = jnp.dot(x, sel_od, preferred_element_type=jnp.float32)
    out_ref[...] = jnp.maximum(ev, od)


def _link_stage(partials, lold, norm_p, kp):
    n = 512
    d2 = 2 * DIM
    return pl.pallas_call(
        _link_body,
        grid=(NB,),
        in_specs=[
            pl.BlockSpec((2, n, DIM), lambda i: (0, i, 0)),
            pl.BlockSpec((n, DIM), lambda i: (i, 0)),
            _full((DIM,)), _full((DIM,)),
            _full((2 * n, n)), _full((2 * n, 1)),
            _full((n, 2 * n)), _full((n, 1)),
            _full((d2,)), _full((d2,)), _full((d2,)), _full((d2,)),
            _full((d2,)), _full((d2,)),
            _full((d2, 2 * d2)), _full((2 * d2,)),
            _full((2 * d2, d2)), _full((d2,)),
        ],
        out_specs=pl.BlockSpec((n, DIM), lambda i: (i, 0)),
        out_shape=jax.ShapeDtypeStruct((NL, DIM), jnp.float32),
    )(partials, lold,
      norm_p["g"], norm_p["b"],
      kp["cu_c1"]["w"], kp["cu_c1"]["b"][:, None],
      kp["cu_c2"]["w"], kp["cu_c2"]["b"][:, None],
      kp["cu_ln"]["g"], kp["cu_ln"]["b"],
      kp["iu_ln"]["g"], kp["iu_ln"]["b"],
      kp["norm"]["g"], kp["norm"]["b"],
      kp["iu_l1"]["w"], kp["iu_l1"]["b"],
      kp["iu_l2"]["w"], kp["iu_l2"]["b"])


def _head_body(pst_ref, lg_ref, ld_ref, w1_ref, b1_ref, w2_ref, b2_ref,
               w3_ref, b3_ref, out_ref):
    h = _ln(pst_ref[...], lg_ref[...], ld_ref[...])
    h = _gelu(jnp.dot(h, w1_ref[...], preferred_element_type=jnp.float32)
              + b1_ref[...])
    h = _gelu(jnp.dot(h, w2_ref[...], preferred_element_type=jnp.float32)
              + b2_ref[...])
    out_ref[...] = jnp.sum(h * w3_ref[...], axis=-1, keepdims=True) + b3_ref[...]


def _head(pst, mh):
    md = 256
    return pl.pallas_call(
        _head_body,
        grid=(NP // PH,),
        in_specs=[
            pl.BlockSpec((PH, DIM), lambda i: (i, 0)),
            _full((DIM,)), _full((DIM,)),
            _full((DIM, md)), _full((md,)),
            _full((md, md)), _full((md,)),
            _full((1, md)), _full((1, 1)),
        ],
        out_specs=pl.BlockSpec((PH, 1), lambda i: (i, 0)),
        out_shape=jax.ShapeDtypeStruct((NP, 1), jnp.float32),
    )(pst, mh["ln"]["g"], mh["ln"]["b"], mh["l1"]["w"], mh["l1"]["b"],
      mh["l2"]["w"], mh["l2"]["b"], mh["l3"]["w"].T, mh["l3"]["b"][None, :])


def _gather(tab, idx_cm):
    return tab[idx_cm]


def _scatter(vals, idx_sc):
    acc = jnp.zeros((ACC, DIM), jnp.float32).at[idx_sc].add(vals)
    return jnp.stack([acc, jnp.zeros_like(acc)], axis=0)


def kernel(bw, tr, p_lidx, mask, params):
    idx = p_lidx.astype(jnp.int32).reshape(NP, L)
    idx_cm = idx.T.reshape(-1)
    idx_sc = (idx_cm + NL) % (NL + 1)
    mask3 = (mask.reshape(NP, S).reshape(NPB, P, S).transpose(0, 2, 1)
             .astype(jnp.float32))

    lemb, pst = _embed(bw, tr, params)
    pad = params["bw_pad"]
    zpad = jnp.zeros((TAB - 1 - NL, DIM), jnp.float32)
    link = lemb
    for lp in params["layers"]:
        tab = jnp.concatenate([pad, link, zpad], axis=0)
        g = _gather(tab, idx_cm)
        g3 = g.reshape(L, NP, DIM)
        pst, pbw = _path_stage(pst, g3, mask3, lp["path"], params["path_w"])
        partials = _scatter(pbw.reshape(E, DIM), idx_sc)
        link = _link_stage(partials, link, params["norm"], lp["link"])
    out = _head(pst, params["mlp_head"])
    return out[:, 0]


# speedup vs baseline: 6.0872x; 2.5527x over previous
import functools

import jax
import jax.numpy as jnp
from jax import lax
from jax.experimental import pallas as pl
from jax.experimental.pallas import tpu as pltpu
from jax.experimental.pallas import tpu_sc as plsc

DIM = 128
L = 20
S = 21
NP = 16384
NL = 2048
NB = 4
TAB = 2056
ACC = 2176
E = NP * L
P = 256
NPB = NP // P
PH = 2048


def _ln(x, g, b, eps=1e-5):
    m = jnp.mean(x, axis=-1, keepdims=True)
    v = jnp.mean((x - m) ** 2, axis=-1, keepdims=True)
    return (x - m) * lax.rsqrt(v + eps) * g + b


def _gelu(x):
    return 0.5 * x * (1.0 + lax.erf(x * 0.7071067811865476))


def _full(shape):
    return pl.BlockSpec(shape, lambda i: tuple(0 for _ in shape))


def _embed_body(bw_ref, tr_ref, wbw_ref, bbw_ref, wtr_ref, btr_ref,
                lemb_ref, pst_ref):
    lemb_ref[...] = bw_ref[...] * wbw_ref[...] + bbw_ref[...]
    pst_ref[...] = tr_ref[...] * wtr_ref[...] + btr_ref[...]


def _embed(bw, tr, params):
    bw2 = bw.reshape(NL, 1)
    tr2 = tr.reshape(NP, 1)
    pe, te = params["bw_emb"], params["tr_emb"]
    return pl.pallas_call(
        _embed_body,
        grid=(8,),
        in_specs=[
            pl.BlockSpec((NL // 8, 1), lambda i: (i, 0)),
            pl.BlockSpec((NP // 8, 1), lambda i: (i, 0)),
            _full((1, DIM)), _full((1, DIM)), _full((1, DIM)), _full((1, DIM)),
        ],
        out_specs=[
            pl.BlockSpec((NL // 8, DIM), lambda i: (i, 0)),
            pl.BlockSpec((NP // 8, DIM), lambda i: (i, 0)),
        ],
        out_shape=[
            jax.ShapeDtypeStruct((NL, DIM), jnp.float32),
            jax.ShapeDtypeStruct((NP, DIM), jnp.float32),
        ],
    )(bw2, tr2, pe["w"], pe["b"][None, :], te["w"], te["b"][None, :])


def _path_body(pst_ref, g_ref, mask_ref,
               wc1_ref, bc1_ref, wc2_ref, bc2_ref,
               g1_ref, d1_ref, g2_ref, d2_ref, g3_ref, d3_ref,
               wl1_ref, bl1_ref, wl2_ref, bl2_ref,
               wpw_ref, bpw_ref,
               pst_out_ref, pbw_out_ref):
    x = jnp.concatenate([pst_ref[...][None], g_ref[...]], axis=0)
    mk = mask_ref[0][:, :, None]
    h = _ln(x * mk, g1_ref[...], d1_ref[...])
    h2 = h.reshape(S, P * DIM)
    hc = jnp.dot(wc1_ref[...], h2, preferred_element_type=jnp.float32)
    hc = _gelu(hc + bc1_ref[...])
    hc = jnp.dot(wc2_ref[...], hc, preferred_element_type=jnp.float32)
    hc = hc + bc2_ref[...]
    x = x + hc.reshape(S, P, DIM)
    h = _ln(x * mk, g2_ref[...], d2_ref[...]).reshape(S * P, DIM)
    h = _gelu(jnp.dot(h, wl1_ref[...], preferred_element_type=jnp.float32)
              + bl1_ref[...])
    h = jnp.dot(h, wl2_ref[...], preferred_element_type=jnp.float32) + bl2_ref[...]
    x = x + h.reshape(S, P, DIM)
    x = _ln(x, g3_ref[...], d3_ref[...])
    p = x[0]
    pst_out_ref[...] = p
    t = jnp.sum(p * wpw_ref[...], axis=-1, keepdims=True) + bpw_ref[...]
    pw = 1.0 / (1.0 + jnp.exp(-t))
    pbw_out_ref[...] = x[1:] * pw[None]


def _path_stage(pst, g3, mask3, pp, pw_p):
    ch = S // 2
    return pl.pallas_call(
        _path_body,
        grid=(NPB,),
        in_specs=[
            pl.BlockSpec((P, DIM), lambda i: (i, 0)),
            pl.BlockSpec((L, P, DIM), lambda i: (0, i, 0)),
            pl.BlockSpec((1, S, P), lambda i: (i, 0, 0)),
            _full((ch, S)), _full((ch, 1)), _full((S, ch)), _full((S, 1)),
            _full((DIM,)), _full((DIM,)), _full((DIM,)), _full((DIM,)),
            _full((DIM,)), _full((DIM,)),
            _full((DIM, DIM // 2)), _full((DIM // 2,)),
            _full((DIM // 2, DIM)), _full((DIM,)),
            _full((1, DIM)), _full((1, 1)),
        ],
        out_specs=[
            pl.BlockSpec((P, DIM), lambda i: (i, 0)),
            pl.BlockSpec((L, P, DIM), lambda i: (0, i, 0)),
        ],
        out_shape=[
            jax.ShapeDtypeStruct((NP, DIM), jnp.float32),
            jax.ShapeDtypeStruct((L, NP, DIM), jnp.float32),
        ],
    )(pst, g3, mask3,
      pp["cu_c1"]["w"], pp["cu_c1"]["b"][:, None],
      pp["cu_c2"]["w"], pp["cu_c2"]["b"][:, None],
      pp["cu_ln"]["g"], pp["cu_ln"]["b"],
      pp["iu_ln"]["g"], pp["iu_ln"]["b"],
      pp["norm"]["g"], pp["norm"]["b"],
      pp["iu_l1"]["w"], pp["iu_l1"]["b"],
      pp["iu_l2"]["w"], pp["iu_l2"]["b"],
      pw_p["w"].T, pw_p["b"][None, :])


def _link_body(part_ref, lold_ref,
               ng_ref, nd_ref,
               wc1_ref, bc1_ref, wc2_ref, bc2_ref,
               g1_ref, d1_ref, g2_ref, d2_ref, g3_ref, d3_ref,
               wl1_ref, bl1_ref, wl2_ref, bl2_ref,
               out_ref):
    tab = part_ref[0] + part_ref[1]
    link = _ln(tab, ng_ref[...], nd_ref[...])
    x = jnp.concatenate([lold_ref[...], link], axis=-1)
    h = _ln(x, g1_ref[...], d1_ref[...])
    hc = _gelu(jnp.dot(wc1_ref[...], h, preferred_element_type=jnp.float32)
               + bc1_ref[...])
    hc = jnp.dot(wc2_ref[...], hc, preferred_element_type=jnp.float32) + bc2_ref[...]
    x = x + hc
    h = _ln(x, g2_ref[...], d2_ref[...])
    h = _gelu(jnp.dot(h, wl1_ref[...], preferred_element_type=jnp.float32)
              + bl1_ref[...])
    h = jnp.dot(h, wl2_ref[...], preferred_element_type=jnp.float32) + bl2_ref[...]
    x = x + h
    x = _ln(x, g3_ref[...], d3_ref[...])
    j = lax.broadcasted_iota(jnp.int32, (2 * DIM, DIM), 0)
    k = lax.broadcasted_iota(jnp.int32, (2 * DIM, DIM), 1)
    sel_ev = (j == 2 * k).astype(jnp.float32)
    sel_od = (j == 2 * k + 1).astype(jnp.float32)
    ev = jnp.dot(x, sel_ev, preferred_element_type=jnp.float32)
    od = jnp.dot(x, sel_od, preferred_element_type=jnp.float32)
    out_ref[...] = jnp.maximum(ev, od)


def _link_stage(partials, lold, norm_p, kp):
    n = 512
    d2 = 2 * DIM
    return pl.pallas_call(
        _link_body,
        grid=(NB,),
        in_specs=[
            pl.BlockSpec((2, n, DIM), lambda i: (0, i, 0)),
            pl.BlockSpec((n, DIM), lambda i: (i, 0)),
            _full((DIM,)), _full((DIM,)),
            _full((2 * n, n)), _full((2 * n, 1)),
            _full((n, 2 * n)), _full((n, 1)),
            _full((d2,)), _full((d2,)), _full((d2,)), _full((d2,)),
            _full((d2,)), _full((d2,)),
            _full((d2, 2 * d2)), _full((2 * d2,)),
            _full((2 * d2, d2)), _full((d2,)),
        ],
        out_specs=pl.BlockSpec((n, DIM), lambda i: (i, 0)),
        out_shape=jax.ShapeDtypeStruct((NL, DIM), jnp.float32),
    )(partials, lold,
      norm_p["g"], norm_p["b"],
      kp["cu_c1"]["w"], kp["cu_c1"]["b"][:, None],
      kp["cu_c2"]["w"], kp["cu_c2"]["b"][:, None],
      kp["cu_ln"]["g"], kp["cu_ln"]["b"],
      kp["iu_ln"]["g"], kp["iu_ln"]["b"],
      kp["norm"]["g"], kp["norm"]["b"],
      kp["iu_l1"]["w"], kp["iu_l1"]["b"],
      kp["iu_l2"]["w"], kp["iu_l2"]["b"])


def _head_body(pst_ref, lg_ref, ld_ref, w1_ref, b1_ref, w2_ref, b2_ref,
               w3_ref, b3_ref, out_ref):
    h = _ln(pst_ref[...], lg_ref[...], ld_ref[...])
    h = _gelu(jnp.dot(h, w1_ref[...], preferred_element_type=jnp.float32)
              + b1_ref[...])
    h = _gelu(jnp.dot(h, w2_ref[...], preferred_element_type=jnp.float32)
              + b2_ref[...])
    out_ref[...] = jnp.sum(h * w3_ref[...], axis=-1, keepdims=True) + b3_ref[...]


def _head(pst, mh):
    md = 256
    return pl.pallas_call(
        _head_body,
        grid=(NP // PH,),
        in_specs=[
            pl.BlockSpec((PH, DIM), lambda i: (i, 0)),
            _full((DIM,)), _full((DIM,)),
            _full((DIM, md)), _full((md,)),
            _full((md, md)), _full((md,)),
            _full((1, md)), _full((1, 1)),
        ],
        out_specs=pl.BlockSpec((PH, 1), lambda i: (i, 0)),
        out_shape=jax.ShapeDtypeStruct((NP, 1), jnp.float32),
    )(pst, mh["ln"]["g"], mh["ln"]["b"], mh["l1"]["w"], mh["l1"]["b"],
      mh["l2"]["w"], mh["l2"]["b"], mh["l3"]["w"].T, mh["l3"]["b"][None, :])


_NC = 2
_NS = 16
_NW = _NC * _NS
_CH = 128
_CHUNKS = E // (_NW * _CH)
_RPS = ACC // _NS


def _sc_mesh():
    return plsc.VectorSubcoreMesh(core_axis_name="c", subcore_axis_name="s")


def _gather_body(tab_hbm, idx_hbm, out_hbm, idx_v, rows_v, sem):
    wid = lax.axis_index("s") * _NC + lax.axis_index("c")
    pltpu.sync_copy(idx_hbm.at[wid], idx_v)
    base = wid * (_CHUNKS * _CH)

    def body(j, carry):
        pltpu.async_copy(tab_hbm.at[idx_v.at[j]], rows_v, sem).wait()
        pltpu.sync_copy(rows_v, out_hbm.at[pl.ds(base + j * _CH, _CH)])
        return carry

    lax.fori_loop(0, _CHUNKS, body, 0)


def _gather(tab, idx3):
    f = functools.partial(
        pl.kernel, mesh=_sc_mesh(),
        out_type=jax.ShapeDtypeStruct((E, DIM), jnp.float32),
        scratch_types=[
            pltpu.VMEM((_CHUNKS, _CH), jnp.int32),
            pltpu.VMEM((_CH, DIM), jnp.float32),
            pltpu.SemaphoreType.DMA,
        ])
    return f(_gather_body)(tab, idx3)


def _scatter_body(vals_hbm, idx_hbm, zeros_hbm, out_hbm, idx_v, vbuf, acc):
    cid = lax.axis_index("c")
    sid = lax.axis_index("s")
    wid = sid * _NC + cid
    pltpu.sync_copy(zeros_hbm.at[pl.ds(sid * _RPS, _RPS)],
                    acc.at[pl.ds(sid * _RPS, _RPS)])
    pltpu.sync_copy(idx_hbm.at[wid], idx_v)
    plsc.subcore_barrier()
    base = wid * (_CHUNKS * _CH)

    def body(j, carry):
        pltpu.sync_copy(vals_hbm.at[pl.ds(base + j * _CH, _CH)], vbuf)
        pltpu.sync_copy(vbuf, acc.at[idx_v.at[j]], add=True)
        return carry

    lax.fori_loop(0, _CHUNKS, body, 0)
    plsc.subcore_barrier()
    pltpu.sync_copy(acc.at[pl.ds(sid * _RPS, _RPS)],
                    out_hbm.at[cid, pl.ds(sid * _RPS, _RPS)])


def _scatter(vals, idx3, zeros):
    f = functools.partial(
        pl.kernel, mesh=_sc_mesh(),
        out_type=jax.ShapeDtypeStruct((_NC, ACC, DIM), jnp.float32),
        scratch_types=[
            pltpu.VMEM((_CHUNKS, _CH), jnp.int32),
            pltpu.VMEM((_CH, DIM), jnp.float32),
            pltpu.VMEM_SHARED((ACC, DIM), jnp.float32),
        ])
    return f(_scatter_body)(vals, idx3, zeros)


def kernel(bw, tr, p_lidx, mask, params):
    idx = p_lidx.astype(jnp.int32).reshape(NP, L)
    idx_cm = idx.T.reshape(-1)
    idx_sc = (idx_cm + NL) % (NL + 1)
    idx_g3 = idx_cm.reshape(_NW, _CHUNKS, _CH)
    idx_s3 = idx_sc.reshape(_NW, _CHUNKS, _CH)
    mask3 = (mask.reshape(NP, S).reshape(NPB, P, S).transpose(0, 2, 1)
             .astype(jnp.float32))

    lemb, pst = _embed(bw, tr, params)
    pad = params["bw_pad"]
    zpad = jnp.zeros((TAB - 1 - NL, DIM), jnp.float32)
    zacc = jnp.zeros((ACC, DIM), jnp.float32)
    link = lemb
    for lp in params["layers"]:
        tab = jnp.concatenate([pad, link, zpad], axis=0)
        g = _gather(tab, idx_g3)
        g3 = g.reshape(L, NP, DIM)
        pst, pbw = _path_stage(pst, g3, mask3, lp["path"], params["path_w"])
        partials = _scatter(pbw.reshape(E, DIM), idx_s3, zacc)
        link = _link_stage(partials, link, params["norm"], lp["link"])
    out = _head(pst, params["mlp_head"])
    return out[:, 0]


# speedup vs baseline: 7.5478x; 1.2399x over previous
import functools

import jax
import jax.numpy as jnp
from jax import lax
from jax.experimental import pallas as pl
from jax.experimental.pallas import tpu as pltpu
from jax.experimental.pallas import tpu_sc as plsc

DIM = 128
L = 20
S = 21
NP = 16384
NL = 2048
NB = 4
TAB = 2176
ACC = 2176
E = NP * L
P = 256
NPB = NP // P
PH = 2048


def _ln(x, g, b, eps=1e-5):
    m = jnp.mean(x, axis=-1, keepdims=True)
    v = jnp.mean((x - m) ** 2, axis=-1, keepdims=True)
    return (x - m) * lax.rsqrt(v + eps) * g + b


def _gelu(x):
    return 0.5 * x * (1.0 + lax.erf(x * 0.7071067811865476))


def _full(shape):
    return pl.BlockSpec(shape, lambda i: tuple(0 for _ in shape))


def _embed_body(bw_ref, tr_ref, wbw_ref, bbw_ref, wtr_ref, btr_ref,
                lemb_ref, pst_ref):
    lemb_ref[...] = bw_ref[...] * wbw_ref[...] + bbw_ref[...]
    pst_ref[...] = tr_ref[...] * wtr_ref[...] + btr_ref[...]


def _embed(bw, tr, params):
    bw2 = bw.reshape(NL, 1)
    tr2 = tr.reshape(NP, 1)
    pe, te = params["bw_emb"], params["tr_emb"]
    return pl.pallas_call(
        _embed_body,
        grid=(8,),
        in_specs=[
            pl.BlockSpec((NL // 8, 1), lambda i: (i, 0)),
            pl.BlockSpec((NP // 8, 1), lambda i: (i, 0)),
            _full((1, DIM)), _full((1, DIM)), _full((1, DIM)), _full((1, DIM)),
        ],
        out_specs=[
            pl.BlockSpec((NL // 8, DIM), lambda i: (i, 0)),
            pl.BlockSpec((NP // 8, DIM), lambda i: (i, 0)),
        ],
        out_shape=[
            jax.ShapeDtypeStruct((NL, DIM), jnp.float32),
            jax.ShapeDtypeStruct((NP, DIM), jnp.float32),
        ],
    )(bw2, tr2, pe["w"], pe["b"][None, :], te["w"], te["b"][None, :])


def _path_body(pst_ref, g_ref, mask_ref,
               wc1_ref, bc1_ref, wc2_ref, bc2_ref,
               g1_ref, d1_ref, g2_ref, d2_ref, g3_ref, d3_ref,
               wl1_ref, bl1_ref, wl2_ref, bl2_ref,
               wpw_ref, bpw_ref,
               pst_out_ref, pbw_out_ref):
    x = jnp.concatenate([pst_ref[...][None], g_ref[...]], axis=0)
    mk = mask_ref[0][:, :, None]
    h = _ln(x * mk, g1_ref[...], d1_ref[...])
    h2 = h.reshape(S, P * DIM)
    hc = jnp.dot(wc1_ref[...], h2, preferred_element_type=jnp.float32)
    hc = _gelu(hc + bc1_ref[...])
    hc = jnp.dot(wc2_ref[...], hc, preferred_element_type=jnp.float32)
    hc = hc + bc2_ref[...]
    x = x + hc.reshape(S, P, DIM)
    h = _ln(x * mk, g2_ref[...], d2_ref[...]).reshape(S * P, DIM)
    h = _gelu(jnp.dot(h, wl1_ref[...], preferred_element_type=jnp.float32)
              + bl1_ref[...])
    h = jnp.dot(h, wl2_ref[...], preferred_element_type=jnp.float32) + bl2_ref[...]
    x = x + h.reshape(S, P, DIM)
    x = _ln(x, g3_ref[...], d3_ref[...])
    p = x[0]
    pst_out_ref[...] = p
    t = jnp.sum(p * wpw_ref[...], axis=-1, keepdims=True) + bpw_ref[...]
    pw = 1.0 / (1.0 + jnp.exp(-t))
    pbw_out_ref[...] = x[1:] * pw[None]


def _path_stage(pst, g3, mask3, pp, pw_p):
    ch = S // 2
    return pl.pallas_call(
        _path_body,
        grid=(NPB,),
        in_specs=[
            pl.BlockSpec((P, DIM), lambda i: (i, 0)),
            pl.BlockSpec((L, P, DIM), lambda i: (0, i, 0)),
            pl.BlockSpec((1, S, P), lambda i: (i, 0, 0)),
            _full((ch, S)), _full((ch, 1)), _full((S, ch)), _full((S, 1)),
            _full((DIM,)), _full((DIM,)), _full((DIM,)), _full((DIM,)),
            _full((DIM,)), _full((DIM,)),
            _full((DIM, DIM // 2)), _full((DIM // 2,)),
            _full((DIM // 2, DIM)), _full((DIM,)),
            _full((1, DIM)), _full((1, 1)),
        ],
        out_specs=[
            pl.BlockSpec((P, DIM), lambda i: (i, 0)),
            pl.BlockSpec((L, P, DIM), lambda i: (0, i, 0)),
        ],
        out_shape=[
            jax.ShapeDtypeStruct((NP, DIM), jnp.float32),
            jax.ShapeDtypeStruct((L, NP, DIM), jnp.float32),
        ],
    )(pst, g3, mask3,
      pp["cu_c1"]["w"], pp["cu_c1"]["b"][:, None],
      pp["cu_c2"]["w"], pp["cu_c2"]["b"][:, None],
      pp["cu_ln"]["g"], pp["cu_ln"]["b"],
      pp["iu_ln"]["g"], pp["iu_ln"]["b"],
      pp["norm"]["g"], pp["norm"]["b"],
      pp["iu_l1"]["w"], pp["iu_l1"]["b"],
      pp["iu_l2"]["w"], pp["iu_l2"]["b"],
      pw_p["w"].T, pw_p["b"][None, :])


def _link_body(part_ref, lold_ref,
               ng_ref, nd_ref,
               wc1_ref, bc1_ref, wc2_ref, bc2_ref,
               g1_ref, d1_ref, g2_ref, d2_ref, g3_ref, d3_ref,
               wl1_ref, bl1_ref, wl2_ref, bl2_ref,
               out_ref):
    tab = part_ref[0] + part_ref[1]
    link = _ln(tab, ng_ref[...], nd_ref[...])
    x = jnp.concatenate([lold_ref[...], link], axis=-1)
    h = _ln(x, g1_ref[...], d1_ref[...])
    hc = _gelu(jnp.dot(wc1_ref[...], h, preferred_element_type=jnp.float32)
               + bc1_ref[...])
    hc = jnp.dot(wc2_ref[...], hc, preferred_element_type=jnp.float32) + bc2_ref[...]
    x = x + hc
    h = _ln(x, g2_ref[...], d2_ref[...])
    h = _gelu(jnp.dot(h, wl1_ref[...], preferred_element_type=jnp.float32)
              + bl1_ref[...])
    h = jnp.dot(h, wl2_ref[...], preferred_element_type=jnp.float32) + bl2_ref[...]
    x = x + h
    x = _ln(x, g3_ref[...], d3_ref[...])
    j = lax.broadcasted_iota(jnp.int32, (2 * DIM, DIM), 0)
    k = lax.broadcasted_iota(jnp.int32, (2 * DIM, DIM), 1)
    sel_ev = (j == 2 * k).astype(jnp.float32)
    sel_od = (j == 2 * k + 1).astype(jnp.float32)
    ev = jnp.dot(x, sel_ev, preferred_element_type=jnp.float32)
    od = jnp.dot(x, sel_od, preferred_element_type=jnp.float32)
    out_ref[...] = jnp.maximum(ev, od)


def _link_stage(partials, lold, norm_p, kp):
    n = 512
    d2 = 2 * DIM
    return pl.pallas_call(
        _link_body,
        grid=(NB,),
        in_specs=[
            pl.BlockSpec((2, n, DIM), lambda i: (0, i, 0)),
            pl.BlockSpec((n, DIM), lambda i: (i, 0)),
            _full((DIM,)), _full((DIM,)),
            _full((2 * n, n)), _full((2 * n, 1)),
            _full((n, 2 * n)), _full((n, 1)),
            _full((d2,)), _full((d2,)), _full((d2,)), _full((d2,)),
            _full((d2,)), _full((d2,)),
            _full((d2, 2 * d2)), _full((2 * d2,)),
            _full((2 * d2, d2)), _full((d2,)),
        ],
        out_specs=pl.BlockSpec((n, DIM), lambda i: (i, 0)),
        out_shape=jax.ShapeDtypeStruct((NL, DIM), jnp.float32),
    )(partials, lold,
      norm_p["g"], norm_p["b"],
      kp["cu_c1"]["w"], kp["cu_c1"]["b"][:, None],
      kp["cu_c2"]["w"], kp["cu_c2"]["b"][:, None],
      kp["cu_ln"]["g"], kp["cu_ln"]["b"],
      kp["iu_ln"]["g"], kp["iu_ln"]["b"],
      kp["norm"]["g"], kp["norm"]["b"],
      kp["iu_l1"]["w"], kp["iu_l1"]["b"],
      kp["iu_l2"]["w"], kp["iu_l2"]["b"])


def _head_body(pst_ref, lg_ref, ld_ref, w1_ref, b1_ref, w2_ref, b2_ref,
               w3_ref, b3_ref, out_ref):
    h = _ln(pst_ref[...], lg_ref[...], ld_ref[...])
    h = _gelu(jnp.dot(h, w1_ref[...], preferred_element_type=jnp.float32)
              + b1_ref[...])
    h = _gelu(jnp.dot(h, w2_ref[...], preferred_element_type=jnp.float32)
              + b2_ref[...])
    out_ref[...] = jnp.sum(h * w3_ref[...], axis=-1, keepdims=True) + b3_ref[...]


def _head(pst, mh):
    md = 256
    return pl.pallas_call(
        _head_body,
        grid=(NP // PH,),
        in_specs=[
            pl.BlockSpec((PH, DIM), lambda i: (i, 0)),
            _full((DIM,)), _full((DIM,)),
            _full((DIM, md)), _full((md,)),
            _full((md, md)), _full((md,)),
            _full((1, md)), _full((1, 1)),
        ],
        out_specs=pl.BlockSpec((PH, 1), lambda i: (i, 0)),
        out_shape=jax.ShapeDtypeStruct((NP, 1), jnp.float32),
    )(pst, mh["ln"]["g"], mh["ln"]["b"], mh["l1"]["w"], mh["l1"]["b"],
      mh["l2"]["w"], mh["l2"]["b"], mh["l3"]["w"].T, mh["l3"]["b"][None, :])


_NC = 2
_NS = 16
_NW = _NC * _NS
_CH = 128
_CHUNKS = E // (_NW * _CH)
_RPS = ACC // _NS


def _sc_mesh():
    return plsc.VectorSubcoreMesh(core_axis_name="c", subcore_axis_name="s")


def _gather_body(tab_hbm, idx_hbm, out_hbm, idx_v, rows0, rows1, stab,
                 sem0, sem1):
    sid = lax.axis_index("s")
    wid = sid * _NC + lax.axis_index("c")
    r = TAB // _NS
    pltpu.sync_copy(tab_hbm.at[pl.ds(sid * r, r)], stab.at[pl.ds(sid * r, r)])
    pltpu.sync_copy(idx_hbm.at[wid], idx_v)
    plsc.subcore_barrier()
    base = wid * (_CHUNKS * _CH)
    bufs, sems = (rows0, rows1), (sem0, sem1)
    pltpu.async_copy(stab.at[idx_v.at[0]], rows0, sem0)

    @pl.loop(0, _CHUNKS, step=2)
    def pair(j):
        for b in range(2):
            jj = j + b
            cur, csem = bufs[b], sems[b]
            nxt, nsem = bufs[1 - b], sems[1 - b]

            @pl.when(jj + 1 < _CHUNKS)
            def _start():
                pltpu.async_copy(stab.at[idx_v.at[jj + 1]], nxt, nsem)

            pltpu.make_async_copy(stab.at[idx_v.at[jj]], cur, csem).wait()
            pltpu.sync_copy(cur, out_hbm.at[pl.ds(base + jj * _CH, _CH)])


def _gather(tab, idx3):
    f = functools.partial(
        pl.kernel, mesh=_sc_mesh(),
        out_type=jax.ShapeDtypeStruct((E, DIM), jnp.float32),
        scratch_types=[
            pltpu.VMEM((_CHUNKS, _CH), jnp.int32),
            pltpu.VMEM((_CH, DIM), jnp.float32),
            pltpu.VMEM((_CH, DIM), jnp.float32),
            pltpu.VMEM_SHARED((TAB, DIM), jnp.float32),
            pltpu.SemaphoreType.DMA,
            pltpu.SemaphoreType.DMA,
        ])
    return f(_gather_body)(tab, idx3)


def _scatter_body(vals_hbm, idx_hbm, zeros_hbm, out_hbm, idx_v, vbuf0, vbuf1,
                  acc, sem0, sem1):
    cid = lax.axis_index("c")
    sid = lax.axis_index("s")
    wid = sid * _NC + cid
    pltpu.sync_copy(zeros_hbm.at[pl.ds(sid * _RPS, _RPS)],
                    acc.at[pl.ds(sid * _RPS, _RPS)])
    pltpu.sync_copy(idx_hbm.at[wid], idx_v)
    plsc.subcore_barrier()
    base = wid * (_CHUNKS * _CH)
    bufs, sems = (vbuf0, vbuf1), (sem0, sem1)
    pltpu.async_copy(vals_hbm.at[pl.ds(base, _CH)], vbuf0, sem0)

    @pl.loop(0, _CHUNKS, step=2)
    def pair(j):
        for b in range(2):
            jj = j + b
            cur, csem = bufs[b], sems[b]
            nxt, nsem = bufs[1 - b], sems[1 - b]

            @pl.when(jj + 1 < _CHUNKS)
            def _start():
                pltpu.async_copy(
                    vals_hbm.at[pl.ds(base + (jj + 1) * _CH, _CH)], nxt, nsem)

            pltpu.make_async_copy(
                vals_hbm.at[pl.ds(base + jj * _CH, _CH)], cur, csem).wait()
            pltpu.sync_copy(cur, acc.at[idx_v.at[jj]], add=True)

    plsc.subcore_barrier()
    pltpu.sync_copy(acc.at[pl.ds(sid * _RPS, _RPS)],
                    out_hbm.at[cid, pl.ds(sid * _RPS, _RPS)])


def _scatter(vals, idx3, zeros):
    f = functools.partial(
        pl.kernel, mesh=_sc_mesh(),
        out_type=jax.ShapeDtypeStruct((_NC, ACC, DIM), jnp.float32),
        scratch_types=[
            pltpu.VMEM((_CHUNKS, _CH), jnp.int32),
            pltpu.VMEM((_CH, DIM), jnp.float32),
            pltpu.VMEM((_CH, DIM), jnp.float32),
            pltpu.VMEM_SHARED((ACC, DIM), jnp.float32),
            pltpu.SemaphoreType.DMA,
            pltpu.SemaphoreType.DMA,
        ])
    return f(_scatter_body)(vals, idx3, zeros)


def kernel(bw, tr, p_lidx, mask, params):
    idx = p_lidx.astype(jnp.int32).reshape(NP, L)
    idx_cm = idx.T.reshape(-1)
    idx_sc = (idx_cm + NL) % (NL + 1)
    idx_g3 = idx_cm.reshape(_NW, _CHUNKS, _CH)
    idx_s3 = idx_sc.reshape(_NW, _CHUNKS, _CH)
    mask3 = (mask.reshape(NP, S).reshape(NPB, P, S).transpose(0, 2, 1)
             .astype(jnp.float32))

    lemb, pst = _embed(bw, tr, params)
    pad = params["bw_pad"]
    zpad = jnp.zeros((TAB - 1 - NL, DIM), jnp.float32)
    zacc = jnp.zeros((ACC, DIM), jnp.float32)
    link = lemb
    for lp in params["layers"]:
        tab = jnp.concatenate([pad, link, zpad], axis=0)
        g = _gather(tab, idx_g3)
        g3 = g.reshape(L, NP, DIM)
        pst, pbw = _path_stage(pst, g3, mask3, lp["path"], params["path_w"])
        partials = _scatter(pbw.reshape(E, DIM), idx_s3, zacc)
        link = _link_stage(partials, link, params["norm"], lp["link"])
    out = _head(pst, params["mlp_head"])
    return out[:, 0]
